# baseline (speedup 1.0000x reference)
import functools

import jax
from jax import lax
import jax.numpy as jnp
from jax.experimental import pallas as pl
from jax.experimental.pallas import tpu as pltpu
from jax.experimental.pallas import tpu_sc as plsc

CB = 8192
D = 256
BETA = 0.25
BLK = 256


def _vq_step(x_ref, cb_ref, c2_ref, rs_ref,
             enc_hbm, idx_ref, loss_ref, perp_ref,
             buf_ref, counts_ref, accx_ref, accd_ref, sem, *, nblk, n_rows):
    i = pl.program_id(0)
    slot = lax.rem(i, 2)

    @pl.when(i == 0)
    def _init():
        counts_ref[...] = jnp.zeros_like(counts_ref)
        accx_ref[...] = jnp.zeros_like(accx_ref)
        accd_ref[...] = jnp.zeros_like(accd_ref)

    @pl.when(i >= 2)
    def _wait_prev():
        pltpu.make_async_copy(
            buf_ref.at[slot], enc_hbm.at[pl.ds((i - 2) * BLK, BLK), :], sem
        ).wait()

    xb = x_ref[...].reshape(D, BLK)
    cb = cb_ref[...]
    c2 = c2_ref[...]
    x2 = jnp.sum(xb * xb, axis=0, keepdims=True).reshape(BLK, 1)

    xc = jax.lax.dot_general(xb, cb, (((0,), (1,)), ((), ())),
                             preferred_element_type=jnp.float32)
    d = (x2 + c2) - 2.0 * xc

    dmin = jnp.min(d, axis=1, keepdims=True)
    col = jax.lax.broadcasted_iota(jnp.int32, (BLK, CB), 1)
    idx = jnp.min(jnp.where(d == dmin, col, CB), axis=1)
    idx_ref[...] = idx.reshape(1, 1, BLK)

    onehot = (col == idx[:, None]).astype(jnp.float32)
    buf_ref[slot] = onehot
    pltpu.make_async_copy(
        buf_ref.at[slot], enc_hbm.at[pl.ds(i * BLK, BLK), :], sem
    ).start()
    ones_row = jnp.ones((1, BLK), jnp.float32)
    counts_ref[...] += jax.lax.dot_general(
        ones_row, onehot, (((1,), (0,)), ((), ())),
        preferred_element_type=jnp.float32)

    accx_ref[...] += jnp.sum(xb, axis=1, keepdims=True)
    accd_ref[...] += jnp.sum(dmin).reshape(1, 1)

    @pl.when(i == nblk - 1)
    def _finish():
        total = jnp.float32(n_rows * D)
        counts = counts_ref[...]
        s1 = jnp.sum(accx_ref[...]) - jnp.sum(counts * rs_ref[...])
        s2 = accd_ref[0, 0]
        loss_ref[...] = (BETA * (s1 / total) + s2 / total).reshape(1, 1)
        e_mean = counts / jnp.float32(n_rows)
        ent = -jnp.sum(e_mean * jnp.log(e_mean + 1e-10))
        perp_ref[...] = jnp.exp(ent).reshape(1, 1)
        pltpu.make_async_copy(
            buf_ref.at[0], enc_hbm.at[pl.ds(0, BLK), :], sem).wait()
        pltpu.make_async_copy(
            buf_ref.at[1], enc_hbm.at[pl.ds(0, BLK), :], sem).wait()


@functools.cache
def _make_sc_gather(n_rows):
    info = plsc.get_sparse_core_info()
    nc, ns = info.num_cores, info.num_subcores
    nw = nc * ns
    rows_per_w = n_rows // nw

    @functools.partial(
        pl.kernel,
        mesh=plsc.VectorSubcoreMesh(core_axis_name="c", subcore_axis_name="s"),
        out_type=jax.ShapeDtypeStruct((n_rows, D), jnp.float32),
        scratch_types=[
            pltpu.VMEM((rows_per_w,), jnp.int32),
            pltpu.VMEM((rows_per_w, D), jnp.float32),
            pltpu.SemaphoreType.DMA,
        ],
    )
    def _sc_gather(table_hbm, idx_hbm, out_hbm, idx_v, rows_v, sem):
        wid = lax.axis_index("s") * nc + lax.axis_index("c")
        base = wid * rows_per_w
        pltpu.sync_copy(idx_hbm.at[pl.ds(base, rows_per_w)], idx_v)
        pltpu.async_copy(table_hbm.at[idx_v], rows_v, sem).wait()
        pltpu.sync_copy(rows_v, out_hbm.at[pl.ds(base, rows_per_w)])

    return _sc_gather


@jax.jit
def kernel(x, codebook):
    b, c, h, w = x.shape
    n = b * h * w
    nblk = n // BLK
    hblk = BLK // w
    c2 = jnp.sum(codebook ** 2, axis=1).reshape(1, CB)
    rs = jnp.sum(codebook, axis=1).reshape(1, CB)

    grid = (nblk,)
    out_shapes = (
        jax.ShapeDtypeStruct((n, CB), jnp.float32),
        jax.ShapeDtypeStruct((nblk, 1, BLK), jnp.int32),
        jax.ShapeDtypeStruct((1, 1), jnp.float32),
        jax.ShapeDtypeStruct((1, 1), jnp.float32),
    )
    enc, idx3, loss, perp = pl.pallas_call(
        functools.partial(_vq_step, nblk=nblk, n_rows=n),
        grid=grid,
        in_specs=[
            pl.BlockSpec((1, D, hblk, w),
                         lambda i, p=(h * w) // BLK: (i // p, 0, i % p, 0)),
            pl.BlockSpec((CB, D), lambda i: (0, 0)),
            pl.BlockSpec((1, CB), lambda i: (0, 0)),
            pl.BlockSpec((1, CB), lambda i: (0, 0)),
        ],
        out_specs=(
            pl.BlockSpec(memory_space=pl.ANY),
            pl.BlockSpec((1, 1, BLK), lambda i: (i, 0, 0)),
            pl.BlockSpec((1, 1), lambda i: (0, 0)),
            pl.BlockSpec((1, 1), lambda i: (0, 0)),
        ),
        out_shape=out_shapes,
        scratch_shapes=[pltpu.VMEM((2, BLK, CB), jnp.float32),
                        pltpu.VMEM((1, CB), jnp.float32),
                        pltpu.VMEM((D, 1), jnp.float32),
                        pltpu.VMEM((1, 1), jnp.float32),
                        pltpu.SemaphoreType.DMA],
    )(x, codebook, c2, rs)

    min_encoding_indices = idx3.reshape(n, 1)
    xqf = _make_sc_gather(n)(codebook, idx3.reshape(n))
    x_quantized = jnp.transpose(xqf.reshape(b, h, w, c), (0, 3, 1, 2))
    return (x_quantized, loss.reshape(()), perp.reshape(()),
            enc, min_encoding_indices)


# speedup vs baseline: 1.0425x; 1.0425x over previous
import functools

import jax
from jax import lax
import jax.numpy as jnp
from jax.experimental import pallas as pl
from jax.experimental.pallas import tpu as pltpu
from jax.experimental.pallas import tpu_sc as plsc

CB = 8192
D = 256
BETA = 0.25
BLK = 512


def _vq_step(x_ref, cb_ref, c2_ref, rs_ref,
             enc_hbm, idx_ref, loss_ref, perp_ref,
             buf_ref, counts_ref, accx_ref, accd_ref, sem, *, nblk, n_rows):
    i = pl.program_id(0)
    slot = lax.rem(i, 2)

    @pl.when(i == 0)
    def _init():
        counts_ref[...] = jnp.zeros_like(counts_ref)
        accx_ref[...] = jnp.zeros_like(accx_ref)
        accd_ref[...] = jnp.zeros_like(accd_ref)

    @pl.when(i >= 2)
    def _wait_prev():
        pltpu.make_async_copy(
            buf_ref.at[slot], enc_hbm.at[pl.ds((i - 2) * BLK, BLK), :], sem
        ).wait()

    xb = x_ref[...].reshape(D, BLK)
    cb = cb_ref[...]
    c2 = c2_ref[...]
    x2 = jnp.sum(xb * xb, axis=0, keepdims=True).reshape(BLK, 1)

    xc = jax.lax.dot_general(xb, cb, (((0,), (1,)), ((), ())),
                             preferred_element_type=jnp.float32)
    d = (x2 + c2) - 2.0 * xc

    dmin = jnp.min(d, axis=1, keepdims=True)
    col = jax.lax.broadcasted_iota(jnp.int32, (BLK, CB), 1)
    idx = jnp.min(jnp.where(d == dmin, col, CB), axis=1)
    idx_ref[...] = idx.reshape(1, 1, BLK)

    onehot = (col == idx[:, None]).astype(jnp.float32)
    buf_ref[slot] = onehot
    pltpu.make_async_copy(
        buf_ref.at[slot], enc_hbm.at[pl.ds(i * BLK, BLK), :], sem
    ).start()
    ones_row = jnp.ones((1, BLK), jnp.float32)
    counts_ref[...] += jax.lax.dot_general(
        ones_row, onehot, (((1,), (0,)), ((), ())),
        preferred_element_type=jnp.float32)

    accx_ref[...] += jnp.sum(xb, axis=1, keepdims=True)
    accd_ref[...] += jnp.sum(dmin).reshape(1, 1)

    @pl.when(i == nblk - 1)
    def _finish():
        total = jnp.float32(n_rows * D)
        counts = counts_ref[...]
        s1 = jnp.sum(accx_ref[...]) - jnp.sum(counts * rs_ref[...])
        s2 = accd_ref[0, 0]
        loss_ref[...] = (BETA * (s1 / total) + s2 / total).reshape(1, 1)
        e_mean = counts / jnp.float32(n_rows)
        ent = -jnp.sum(e_mean * jnp.log(e_mean + 1e-10))
        perp_ref[...] = jnp.exp(ent).reshape(1, 1)
        pltpu.make_async_copy(
            buf_ref.at[0], enc_hbm.at[pl.ds(0, BLK), :], sem).wait()
        pltpu.make_async_copy(
            buf_ref.at[1], enc_hbm.at[pl.ds(0, BLK), :], sem).wait()


@functools.cache
def _make_sc_gather(n_rows):
    info = plsc.get_sparse_core_info()
    nc, ns = info.num_cores, info.num_subcores
    nw = nc * ns
    rows_per_w = n_rows // nw

    @functools.partial(
        pl.kernel,
        mesh=plsc.VectorSubcoreMesh(core_axis_name="c", subcore_axis_name="s"),
        out_type=jax.ShapeDtypeStruct((n_rows, D), jnp.float32),
        scratch_types=[
            pltpu.VMEM((rows_per_w,), jnp.int32),
            pltpu.VMEM((rows_per_w, D), jnp.float32),
            pltpu.SemaphoreType.DMA,
        ],
    )
    def _sc_gather(table_hbm, idx_hbm, out_hbm, idx_v, rows_v, sem):
        wid = lax.axis_index("s") * nc + lax.axis_index("c")
        base = wid * rows_per_w
        pltpu.sync_copy(idx_hbm.at[pl.ds(base, rows_per_w)], idx_v)
        pltpu.async_copy(table_hbm.at[idx_v], rows_v, sem).wait()
        pltpu.sync_copy(rows_v, out_hbm.at[pl.ds(base, rows_per_w)])

    return _sc_gather


@jax.jit
def kernel(x, codebook):
    b, c, h, w = x.shape
    n = b * h * w
    nblk = n // BLK
    hblk = BLK // w
    c2 = jnp.sum(codebook ** 2, axis=1).reshape(1, CB)
    rs = jnp.sum(codebook, axis=1).reshape(1, CB)

    grid = (nblk,)
    out_shapes = (
        jax.ShapeDtypeStruct((n, CB), jnp.float32),
        jax.ShapeDtypeStruct((nblk, 1, BLK), jnp.int32),
        jax.ShapeDtypeStruct((1, 1), jnp.float32),
        jax.ShapeDtypeStruct((1, 1), jnp.float32),
    )
    enc, idx3, loss, perp = pl.pallas_call(
        functools.partial(_vq_step, nblk=nblk, n_rows=n),
        grid=grid,
        in_specs=[
            pl.BlockSpec((1, D, hblk, w),
                         lambda i, p=(h * w) // BLK: (i // p, 0, i % p, 0)),
            pl.BlockSpec((CB, D), lambda i: (0, 0)),
            pl.BlockSpec((1, CB), lambda i: (0, 0)),
            pl.BlockSpec((1, CB), lambda i: (0, 0)),
        ],
        out_specs=(
            pl.BlockSpec(memory_space=pl.ANY),
            pl.BlockSpec((1, 1, BLK), lambda i: (i, 0, 0)),
            pl.BlockSpec((1, 1), lambda i: (0, 0)),
            pl.BlockSpec((1, 1), lambda i: (0, 0)),
        ),
        out_shape=out_shapes,
        scratch_shapes=[pltpu.VMEM((2, BLK, CB), jnp.float32),
                        pltpu.VMEM((1, CB), jnp.float32),
                        pltpu.VMEM((D, 1), jnp.float32),
                        pltpu.VMEM((1, 1), jnp.float32),
                        pltpu.SemaphoreType.DMA],
        compiler_params=pltpu.CompilerParams(
            vmem_limit_bytes=100 * 1024 * 1024),
    )(x, codebook, c2, rs)

    min_encoding_indices = idx3.reshape(n, 1)
    xqf = _make_sc_gather(n)(codebook, idx3.reshape(n))
    x_quantized = jnp.transpose(xqf.reshape(b, h, w, c), (0, 3, 1, 2))
    return (x_quantized, loss.reshape(()), perp.reshape(()),
            enc, min_encoding_indices)


# speedup vs baseline: 1.1970x; 1.1482x over previous
import functools

import jax
from jax import lax
import jax.numpy as jnp
from jax.experimental import pallas as pl
from jax.experimental.pallas import tpu as pltpu
from jax.experimental.pallas import tpu_sc as plsc

CB = 8192
D = 256
BETA = 0.25
BLK = 512


def _vq_step(x_ref, cb_ref, c2_ref, rs_ref,
             enc_hbm, idx_ref, loss_ref, perp_ref,
             buf_ref, counts_ref, accx_ref, accd_ref, sem, *, nblk, n_rows):
    i = pl.program_id(0)
    slot = lax.rem(i, 2)

    @pl.when(i == 0)
    def _init():
        counts_ref[...] = jnp.zeros_like(counts_ref)
        accx_ref[...] = jnp.zeros_like(accx_ref)
        accd_ref[...] = jnp.zeros_like(accd_ref)

    @pl.when(i >= 2)
    def _wait_prev():
        pltpu.make_async_copy(
            buf_ref.at[slot], enc_hbm.at[pl.ds((i - 2) * BLK, BLK), :], sem
        ).wait()

    himg = x_ref.shape[2]
    r = lax.rem(i, himg // BLK)
    xb = x_ref[0, :, pl.ds(r * BLK, BLK)]
    cb = cb_ref[...]
    c2 = c2_ref[...]
    x2 = jnp.sum(xb * xb, axis=0, keepdims=True).reshape(BLK, 1)

    xc = jax.lax.dot_general(xb, cb, (((0,), (1,)), ((), ())),
                             preferred_element_type=jnp.float32)
    d = (x2 + c2) - 2.0 * xc

    dmin = jnp.min(d, axis=1, keepdims=True)
    col = jax.lax.broadcasted_iota(jnp.int32, (BLK, CB), 1)
    idx = jnp.min(jnp.where(d == dmin, col, CB), axis=1)
    idx_ref[...] = idx.reshape(1, 1, BLK)

    onehot = (col == idx[:, None]).astype(jnp.float32)
    buf_ref[slot] = onehot
    pltpu.make_async_copy(
        buf_ref.at[slot], enc_hbm.at[pl.ds(i * BLK, BLK), :], sem
    ).start()
    ones_row = jnp.ones((1, BLK), jnp.float32)
    counts_ref[...] += jax.lax.dot_general(
        ones_row, onehot, (((1,), (0,)), ((), ())),
        preferred_element_type=jnp.float32)

    accx_ref[...] += jnp.sum(xb, axis=1, keepdims=True)
    accd_ref[...] += jnp.sum(dmin).reshape(1, 1)

    @pl.when(i == nblk - 1)
    def _finish():
        total = jnp.float32(n_rows * D)
        counts = counts_ref[...]
        s1 = jnp.sum(accx_ref[...]) - jnp.sum(counts * rs_ref[...])
        s2 = accd_ref[0, 0]
        loss_ref[...] = (BETA * (s1 / total) + s2 / total).reshape(1, 1)
        e_mean = counts / jnp.float32(n_rows)
        ent = -jnp.sum(e_mean * jnp.log(e_mean + 1e-10))
        perp_ref[...] = jnp.exp(ent).reshape(1, 1)
        pltpu.make_async_copy(
            buf_ref.at[0], enc_hbm.at[pl.ds(0, BLK), :], sem).wait()
        pltpu.make_async_copy(
            buf_ref.at[1], enc_hbm.at[pl.ds(0, BLK), :], sem).wait()


@functools.cache
def _make_sc_gather(n_rows):
    info = plsc.get_sparse_core_info()
    nc, ns = info.num_cores, info.num_subcores
    nw = nc * ns
    rows_per_w = n_rows // nw

    @functools.partial(
        pl.kernel,
        mesh=plsc.VectorSubcoreMesh(core_axis_name="c", subcore_axis_name="s"),
        out_type=jax.ShapeDtypeStruct((n_rows, D), jnp.float32),
        scratch_types=[
            pltpu.VMEM((rows_per_w,), jnp.int32),
            pltpu.VMEM((rows_per_w, D), jnp.float32),
            pltpu.SemaphoreType.DMA,
        ],
    )
    def _sc_gather(table_hbm, idx_hbm, out_hbm, idx_v, rows_v, sem):
        wid = lax.axis_index("s") * nc + lax.axis_index("c")
        base = wid * rows_per_w
        pltpu.sync_copy(idx_hbm.at[pl.ds(base, rows_per_w)], idx_v)
        pltpu.async_copy(table_hbm.at[idx_v], rows_v, sem).wait()
        pltpu.sync_copy(rows_v, out_hbm.at[pl.ds(base, rows_per_w)])

    return _sc_gather


@jax.jit
def kernel(x, codebook):
    b, c, h, w = x.shape
    n = b * h * w
    nblk = n // BLK
    hblk = BLK // w
    c2 = jnp.sum(codebook ** 2, axis=1).reshape(1, CB)
    rs = jnp.sum(codebook, axis=1).reshape(1, CB)

    grid = (nblk,)
    out_shapes = (
        jax.ShapeDtypeStruct((n, CB), jnp.float32),
        jax.ShapeDtypeStruct((nblk, 1, BLK), jnp.int32),
        jax.ShapeDtypeStruct((1, 1), jnp.float32),
        jax.ShapeDtypeStruct((1, 1), jnp.float32),
    )
    enc, idx3, loss, perp = pl.pallas_call(
        functools.partial(_vq_step, nblk=nblk, n_rows=n),
        grid=grid,
        in_specs=[
            pl.BlockSpec((1, D, h * w),
                         lambda i, p=(h * w) // BLK: (i // p, 0, 0)),
            pl.BlockSpec((CB, D), lambda i: (0, 0)),
            pl.BlockSpec((1, CB), lambda i: (0, 0)),
            pl.BlockSpec((1, CB), lambda i: (0, 0)),
        ],
        out_specs=(
            pl.BlockSpec(memory_space=pl.ANY),
            pl.BlockSpec((1, 1, BLK), lambda i: (i, 0, 0)),
            pl.BlockSpec((1, 1), lambda i: (0, 0)),
            pl.BlockSpec((1, 1), lambda i: (0, 0)),
        ),
        out_shape=out_shapes,
        scratch_shapes=[pltpu.VMEM((2, BLK, CB), jnp.float32),
                        pltpu.VMEM((1, CB), jnp.float32),
                        pltpu.VMEM((D, 1), jnp.float32),
                        pltpu.VMEM((1, 1), jnp.float32),
                        pltpu.SemaphoreType.DMA],
        compiler_params=pltpu.CompilerParams(
            vmem_limit_bytes=100 * 1024 * 1024),
    )(x.reshape(b, c, h * w), codebook, c2, rs)

    min_encoding_indices = idx3.reshape(n, 1)
    xqf = _make_sc_gather(n)(codebook, idx3.reshape(n))
    x_quantized = jnp.transpose(xqf.reshape(b, h, w, c), (0, 3, 1, 2))
    return (x_quantized, loss.reshape(()), perp.reshape(()),
            enc, min_encoding_indices)
